# baseline (speedup 1.0000x reference)
import functools

import jax
import jax.numpy as jnp
from jax import lax
from jax.experimental import pallas as pl
from jax.experimental.pallas import tpu as pltpu
from jax.experimental.pallas import tpu_sc as plsc

NC = 2
NS = 16
NW = NC * NS
L = 16

K = 128
SUP = 8
ACC_ROWS = 10112
CNT_N = 10240


@functools.lru_cache(maxsize=None)
def _make_agg_kernel(n, d, e_pad, with_cnt):
  epw = e_pad // NW
  n_chunks = epw // K
  n_super = n_chunks // SUP
  zpw = ACC_ROWS // NS

  mesh = plsc.VectorSubcoreMesh(core_axis_name="c", subcore_axis_name="s",
                                num_cores=NC, num_subcores=NS)
  out_type = [jax.ShapeDtypeStruct((NC, ACC_ROWS, d), jnp.float32)]
  scratch = [
      pltpu.VMEM((SUP, K), jnp.int32),
      pltpu.VMEM((SUP, K), jnp.int32),
      pltpu.VMEM((K, d), jnp.float32),
      pltpu.VMEM((K, d), jnp.float32),
      pltpu.SemaphoreType.DMA,
      pltpu.SemaphoreType.DMA,
      pltpu.SemaphoreType.DMA,
      pltpu.SemaphoreType.DMA,
      pltpu.VMEM_SHARED((ACC_ROWS, d), jnp.float32),
  ]
  if with_cnt:
    out_type.append(jax.ShapeDtypeStruct((NC, NS, CNT_N), jnp.float32))
    scratch.append(pltpu.VMEM((CNT_N,), jnp.float32))

  def body(h_hbm, src_hbm, dst_hbm, zero_hbm, out_agg, *rest):
    if with_cnt:
      (out_cnt, src_v, dst_v, rows0, rows1, gs0, gs1, ss0, ss1,
       acc, cnt_l) = rest
    else:
      (src_v, dst_v, rows0, rows1, gs0, gs1, ss0, ss1, acc) = rest
    rows = (rows0, rows1)
    gsem = (gs0, gs1)
    ssem = (ss0, ss1)
    c = lax.axis_index("c")
    s = lax.axis_index("s")
    wid = s * NC + c

    z_off = pl.multiple_of(s * zpw, 8)
    pltpu.sync_copy(zero_hbm.at[pl.ds(z_off, zpw)], acc.at[pl.ds(z_off, zpw)])
    if with_cnt:
      zv = jnp.zeros((L,), jnp.float32)
      def zero_cnt(i, carry):
        cnt_l[pl.ds(i * L, L)] = zv
        return carry
      lax.fori_loop(0, CNT_N // L, zero_cnt, 0)
    plsc.subcore_barrier()

    base = wid * n_chunks
    ones = jnp.full((L,), 1.0, jnp.float32)

    def super_step(i, carry):
      row0 = pl.multiple_of(base + i * SUP, SUP)
      pltpu.sync_copy(src_hbm.at[pl.ds(row0, SUP)], src_v)
      pltpu.sync_copy(dst_hbm.at[pl.ds(row0, SUP)], dst_v)
      waits = [None, None]
      for k in range(SUP):
        b = k % 2
        if waits[b] is not None:
          waits[b].wait()
        g = pltpu.async_copy(h_hbm.at[src_v.at[k]], rows[b], gsem[b])
        if with_cnt:
          for j in range(K // L):
            dv = dst_v[k, pl.ds(j * L, L)]
            plsc.addupdate_scatter(cnt_l, [dv], ones)
        g.wait()
        waits[b] = pltpu.async_copy(rows[b], acc.at[dst_v.at[k]],
                                    ssem[b], add=True)
      waits[0].wait()
      waits[1].wait()
      return carry
    lax.fori_loop(0, n_super, super_step, 0)
    plsc.subcore_barrier()

    pltpu.sync_copy(acc.at[pl.ds(z_off, zpw)],
                    out_agg.at[c, pl.ds(z_off, zpw)])
    if with_cnt:
      pltpu.sync_copy(cnt_l, out_cnt.at[c, s])

  return pl.kernel(
      body, out_type=out_type, mesh=mesh, scratch_types=scratch,
      compiler_params=pltpu.CompilerParams(needs_layout_passes=False))


def _cnt_reduce_tc(cnt_parts):
  m, w = cnt_parts.shape

  def body(c_ref, o_ref):
    o_ref[...] = jnp.sum(c_ref[...], axis=0, keepdims=True)

  return pl.pallas_call(
      body,
      in_specs=[pl.BlockSpec((m, w), lambda: (0, 0))],
      out_specs=pl.BlockSpec((1, w), lambda: (0, 0)),
      out_shape=jax.ShapeDtypeStruct((1, w), jnp.float32),
  )(cnt_parts)


def _layer_tc(p0, p1, c3d, h, Wl, bl, Wr, relu, resid):
  n, d = h.shape
  R = 400
  grid = (n // R,)

  def body(p0_ref, p1_ref, c_ref, h_ref, wl_ref, bl_ref, wr_ref, *rest):
    if resid is not None:
      x_ref, o_ref = rest
    else:
      (o_ref,) = rest
    p = p0_ref[...] + p1_ref[...]
    cb = c_ref[0]
    ones_row = jnp.ones((1, d), jnp.float32)
    cc = lax.dot_general(cb, ones_row, (((0,), (0,)), ((), ())),
                         preferred_element_type=jnp.float32)
    aggm = p / jnp.maximum(cc, 1.0)
    y = lax.dot_general(aggm, wl_ref[...], (((1,), (1,)), ((), ())),
                        preferred_element_type=jnp.float32)
    y = y + lax.dot_general(h_ref[...], wr_ref[...], (((1,), (1,)), ((), ())),
                            preferred_element_type=jnp.float32)
    y = y + bl_ref[...]
    if relu:
      y = jnp.maximum(y, 0.0)
    if resid is not None:
      y = y + x_ref[...]
    o_ref[...] = y

  in_specs = [
      pl.BlockSpec((R, d), lambda j: (j, 0)),
      pl.BlockSpec((R, d), lambda j: (j, 0)),
      pl.BlockSpec((1, 1, R), lambda j: (j, 0, 0)),
      pl.BlockSpec((R, d), lambda j: (j, 0)),
      pl.BlockSpec((d, d), lambda j: (0, 0)),
      pl.BlockSpec((1, d), lambda j: (0, 0)),
      pl.BlockSpec((d, d), lambda j: (0, 0)),
  ]
  args = [p0, p1, c3d, h, Wl, bl.reshape(1, d), Wr]
  if resid is not None:
    in_specs.append(pl.BlockSpec((R, d), lambda j: (j, 0)))
    args.append(resid)
  return pl.pallas_call(
      body, grid=grid, in_specs=in_specs,
      out_specs=pl.BlockSpec((R, d), lambda j: (j, 0)),
      out_shape=jax.ShapeDtypeStruct((n, d), jnp.float32),
  )(*args)


def kernel(x, edge_index, Wl0, bl0, Wr0, Wl1, bl1, Wr1):
  n, d = x.shape
  e = edge_index.shape[1]
  grain = NW * K * SUP
  e_pad = -(-e // grain) * grain
  pad = e_pad - e
  src = jnp.concatenate([edge_index[0], jnp.zeros((pad,), jnp.int32)])
  dst = jnp.concatenate([edge_index[1], jnp.full((pad,), n, jnp.int32)])
  src2d = src.reshape(-1, K)
  dst2d = dst.reshape(-1, K)
  zeros_hbm = jnp.zeros((ACC_ROWS, d), jnp.float32)

  agg0_k = _make_agg_kernel(n, d, e_pad, True)
  agg1_k = _make_agg_kernel(n, d, e_pad, False)

  part0, cnt = agg0_k(x, src2d, dst2d, zeros_hbm)
  part0 = part0[:, :n]
  cnt_sum = _cnt_reduce_tc(cnt.reshape(NC * NS, CNT_N))
  c3d = cnt_sum[0, :n].reshape(-1, 1, 400)

  h1 = _layer_tc(part0[0], part0[1], c3d, x, Wl0, bl0, Wr0,
                 relu=True, resid=None)
  part1 = agg1_k(h1, src2d, dst2d, zeros_hbm)
  if isinstance(part1, (list, tuple)):
    part1 = part1[0]
  part1 = part1[:, :n]
  out = _layer_tc(part1[0], part1[1], c3d, h1, Wl1, bl1, Wr1,
                  relu=False, resid=x)
  return out
